# Initial kernel scaffold; baseline (speedup 1.0000x reference)
import jax
import jax.numpy as jnp
from jax.experimental import pallas as pl


def kernel(x, pos_table):
    raise NotImplementedError("write your pallas kernel here")


# speedup vs baseline: 1.9247x; 1.9247x over previous
import jax
import jax.numpy as jnp
from jax.experimental import pallas as pl


def _add_pos_kernel(x_ref, pos_ref, out_ref):
    out_ref[...] = x_ref[...] + pos_ref[...][None, :, :]


def kernel(x, pos_table):
    batch, seq_len, d_model = x.shape
    blk = 256
    grid = (seq_len // blk,)
    return pl.pallas_call(
        _add_pos_kernel,
        grid=grid,
        in_specs=[
            pl.BlockSpec((batch, blk, d_model), lambda s: (0, s, 0)),
            pl.BlockSpec((blk, d_model), lambda s: (s, 0)),
        ],
        out_specs=pl.BlockSpec((batch, blk, d_model), lambda s: (0, s, 0)),
        out_shape=jax.ShapeDtypeStruct((batch, seq_len, d_model), x.dtype),
    )(x, pos_table)


# speedup vs baseline: 1.9567x; 1.0166x over previous
import jax
import jax.numpy as jnp
from jax.experimental import pallas as pl


def _add_pos_kernel(x_ref, pos_ref, out_ref):
    out_ref[...] = x_ref[...] + pos_ref[...][None, :, :]


def kernel(x, pos_table):
    batch, seq_len, d_model = x.shape
    blk = 512
    grid = (seq_len // blk,)
    return pl.pallas_call(
        _add_pos_kernel,
        grid=grid,
        in_specs=[
            pl.BlockSpec((batch, blk, d_model), lambda s: (0, s, 0)),
            pl.BlockSpec((blk, d_model), lambda s: (s, 0)),
        ],
        out_specs=pl.BlockSpec((batch, blk, d_model), lambda s: (0, s, 0)),
        out_shape=jax.ShapeDtypeStruct((batch, seq_len, d_model), x.dtype),
    )(x, pos_table)
